# Initial kernel scaffold; baseline (speedup 1.0000x reference)
import jax
import jax.numpy as jnp
from jax.experimental import pallas as pl


def kernel(group0, edge_index, group_mask, edge_attr, W_prep, b_prep, W_tube, b_tube, W_e1, b_e1, W_em, b_em, W_root_m, bias_m, W_ea, b_ea, W_root_a, bias_a, W_out, b_out):
    raise NotImplementedError("write your pallas kernel here")


# speedup vs baseline: 5.0474x; 5.0474x over previous
import functools

import jax
import jax.numpy as jnp
from jax import lax
from jax.experimental import pallas as pl
from jax.experimental.pallas import tpu as pltpu
from jax.experimental.pallas import tpu_sc as plsc

N_NODES = 10000
N_EDGES = 160000
NC = 2
NS = 16
NW = NC * NS
EDGES_PER_TILE = N_EDGES // NW
CHUNK = 125
CHUNKS_PER_TILE = EDGES_PER_TILE // CHUNK
STAGE = 1000
ROWS_PER_TILE = N_NODES // NS

_mesh = plsc.VectorSubcoreMesh(core_axis_name="c", subcore_axis_name="s")


def _prep_body(g0, mask, Wp, bp, Wt, bt, Wr, br, x_out, root_out):
    prep = jnp.dot(g0[...], Wp[...], preferred_element_type=jnp.float32) + bp[...]
    x0 = jnp.maximum(jnp.where(mask[...] == 0, prep, 0.0), 0.0)
    x = jnp.maximum(
        jnp.dot(x0, Wt[...], preferred_element_type=jnp.float32) + bt[...], 0.0)
    x_out[...] = x
    root_out[...] = (
        jnp.dot(x, Wr[...], preferred_element_type=jnp.float32) + br[...])


def _edge_body(ea, xs, We, be, R, T, W2, B2, out):
    h = jax.nn.sigmoid(
        jnp.dot(ea[...], We[...], preferred_element_type=jnp.float32) + be[...])
    x = xs[...]
    z = (jnp.dot(h, R[...], preferred_element_type=jnp.float32)
         * jnp.dot(x, T[...], preferred_element_type=jnp.float32))
    msg = (jnp.dot(z, W2[...], preferred_element_type=jnp.float32)
           + jnp.dot(x, B2[...], preferred_element_type=jnp.float32))
    b = msg.shape[0]
    pad = jnp.concatenate(
        [jnp.ones((b, 1), jnp.float32), jnp.zeros((b, 7), jnp.float32)], axis=1)
    out[...] = jnp.concatenate([msg, pad], axis=1)


def _final_body(p, root, Wo, bo, y_out):
    s = p[0] + p[1]
    cnt = jnp.maximum(s[:, 24:25], 1.0)
    mean_m = s[:, :16] / cnt
    h = jnp.concatenate(
        [mean_m + root[:, :16], s[:, 16:24] + root[:, 16:24]], axis=1)
    h = jnp.maximum(h, 0.0)
    y_out[...] = jnp.dot(h, Wo[...], preferred_element_type=jnp.float32) + bo[...]


@functools.partial(
    pl.kernel,
    mesh=_mesh,
    out_type=jax.ShapeDtypeStruct((N_EDGES, 16), jnp.float32),
    scratch_types=[
        pltpu.VMEM((CHUNKS_PER_TILE, CHUNK), jnp.int32),
        pltpu.VMEM((EDGES_PER_TILE, 16), jnp.float32),
        pltpu.SemaphoreType.DMA,
    ],
    compiler_params=pltpu.CompilerParams(use_tc_tiling_on_sc=False),
)
def _sc_gather(src_idx_hbm, x_hbm, out_hbm, idx_v, rows_v, sem):
    c = lax.axis_index("c")
    s = lax.axis_index("s")
    wid = c * NS + s
    pltpu.sync_copy(src_idx_hbm.at[wid], idx_v)

    def body(j, carry):
        pltpu.async_copy(
            x_hbm.at[idx_v.at[j]], rows_v.at[pl.ds(j * CHUNK, CHUNK)], sem
        ).wait()
        return carry

    lax.fori_loop(0, CHUNKS_PER_TILE, body, 0)
    pltpu.sync_copy(rows_v, out_hbm.at[pl.ds(wid * EDGES_PER_TILE,
                                             EDGES_PER_TILE)])


@functools.partial(
    pl.kernel,
    mesh=_mesh,
    out_type=jax.ShapeDtypeStruct((NC, N_NODES, 32), jnp.float32),
    scratch_types=[
        pltpu.VMEM((CHUNKS_PER_TILE, CHUNK), jnp.int32),
        pltpu.VMEM((STAGE, 32), jnp.float32),
        pltpu.VMEM_SHARED((N_NODES, 32), jnp.float32),
    ],
    compiler_params=pltpu.CompilerParams(use_tc_tiling_on_sc=False),
)
def _sc_scatter(dst_idx_hbm, msg_hbm, zeros_hbm, out_hbm, idx_v, msg_v, accum):
    c = lax.axis_index("c")
    s = lax.axis_index("s")
    wid = c * NS + s
    pltpu.sync_copy(zeros_hbm.at[pl.ds(s * ROWS_PER_TILE, ROWS_PER_TILE)],
                    accum.at[pl.ds(s * ROWS_PER_TILE, ROWS_PER_TILE)])
    pltpu.sync_copy(dst_idx_hbm.at[wid], idx_v)
    plsc.subcore_barrier()

    base = wid * EDGES_PER_TILE
    inner_n = STAGE // CHUNK

    def outer(k, carry):
        pltpu.sync_copy(msg_hbm.at[pl.ds(base + k * STAGE, STAGE)], msg_v)

        def inner(j, carry2):
            pltpu.sync_copy(msg_v.at[pl.ds(j * CHUNK, CHUNK)],
                            accum.at[idx_v.at[k * inner_n + j]], add=True)
            return carry2

        lax.fori_loop(0, inner_n, inner, 0)
        return carry

    lax.fori_loop(0, EDGES_PER_TILE // STAGE, outer, 0)
    plsc.subcore_barrier()
    pltpu.sync_copy(accum.at[pl.ds(s * ROWS_PER_TILE, ROWS_PER_TILE)],
                    out_hbm.at[c, pl.ds(s * ROWS_PER_TILE, ROWS_PER_TILE)])


def kernel(group0, edge_index, group_mask, edge_attr,
           W_prep, b_prep, W_tube, b_tube,
           W_e1, b_e1,
           W_em, b_em, W_root_m, bias_m,
           W_ea, b_ea, W_root_a, bias_a,
           W_out, b_out):
    f32 = jnp.float32
    mask2d = group_mask.astype(jnp.int32).reshape(N_NODES, 1)
    src_idx = edge_index[0].astype(jnp.int32).reshape(NW, CHUNKS_PER_TILE, CHUNK)
    dst_idx = edge_index[1].astype(jnp.int32).reshape(NW, CHUNKS_PER_TILE, CHUNK)

    Wr = jnp.concatenate([W_root_m, W_root_a], axis=1)
    br = jnp.concatenate([bias_m, bias_a]).reshape(1, 24)
    W2 = jnp.concatenate([W_em.reshape(16, 16, 16).reshape(256, 16),
                          W_ea.reshape(16, 16, 8).reshape(256, 8)], axis=1)
    B2 = jnp.concatenate([b_em.reshape(16, 16), b_ea.reshape(16, 8)], axis=1)
    eye = jnp.eye(16, dtype=f32)
    R = jnp.repeat(eye, 16, axis=1)
    T = jnp.tile(eye, (1, 16))

    nblk = 2000
    full = lambda i: (0, 0)
    x, root = pl.pallas_call(
        _prep_body,
        grid=(N_NODES // nblk,),
        in_specs=[
            pl.BlockSpec((nblk, 128), lambda i: (i, 0)),
            pl.BlockSpec((nblk, 1), lambda i: (i, 0)),
            pl.BlockSpec((128, 64), full),
            pl.BlockSpec((1, 64), full),
            pl.BlockSpec((64, 16), full),
            pl.BlockSpec((1, 16), full),
            pl.BlockSpec((16, 24), full),
            pl.BlockSpec((1, 24), full),
        ],
        out_specs=[
            pl.BlockSpec((nblk, 16), lambda i: (i, 0)),
            pl.BlockSpec((nblk, 24), lambda i: (i, 0)),
        ],
        out_shape=[
            jax.ShapeDtypeStruct((N_NODES, 16), f32),
            jax.ShapeDtypeStruct((N_NODES, 24), f32),
        ],
    )(group0, mask2d, W_prep, b_prep.reshape(1, 64),
      W_tube, b_tube.reshape(1, 16), Wr, br)

    x_src = _sc_gather(src_idx, x)

    eblk = 2000
    msg = pl.pallas_call(
        _edge_body,
        grid=(N_EDGES // eblk,),
        in_specs=[
            pl.BlockSpec((eblk, 4), lambda i: (i, 0)),
            pl.BlockSpec((eblk, 16), lambda i: (i, 0)),
            pl.BlockSpec((4, 16), full),
            pl.BlockSpec((1, 16), full),
            pl.BlockSpec((16, 256), full),
            pl.BlockSpec((16, 256), full),
            pl.BlockSpec((256, 24), full),
            pl.BlockSpec((16, 24), full),
        ],
        out_specs=pl.BlockSpec((eblk, 32), lambda i: (i, 0)),
        out_shape=jax.ShapeDtypeStruct((N_EDGES, 32), f32),
    )(edge_attr, x_src, W_e1, b_e1.reshape(1, 16), R, T, W2, B2)

    zeros_init = jnp.zeros((N_NODES, 32), f32)
    partials = _sc_scatter(dst_idx, msg, zeros_init)

    y = pl.pallas_call(
        _final_body,
        grid=(N_NODES // nblk,),
        in_specs=[
            pl.BlockSpec((NC, nblk, 32), lambda i: (0, i, 0)),
            pl.BlockSpec((nblk, 24), lambda i: (i, 0)),
            pl.BlockSpec((24, 2), full),
            pl.BlockSpec((1, 2), full),
        ],
        out_specs=pl.BlockSpec((nblk, 2), lambda i: (i, 0)),
        out_shape=jax.ShapeDtypeStruct((N_NODES, 2), f32),
    )(partials, root, W_out, b_out.reshape(1, 2))
    return y
